# baseline (speedup 1.0000x reference)
import functools

import jax
import jax.numpy as jnp
from jax import lax
from jax.experimental import pallas as pl
from jax.experimental.pallas import tpu as pltpu
from jax.experimental.pallas import tpu_sc as plsc

_NC = 2
_NS = 16
_NW = _NC * _NS


def _sc_gather_t(uflat, iflat, idxu, idxi, du, di):
  B = idxu.shape[1]
  bw = B // _NW

  mesh = plsc.VectorSubcoreMesh(core_axis_name="c", subcore_axis_name="s")

  @functools.partial(
      pl.kernel,
      out_type=[
          jax.ShapeDtypeStruct((du, B), jnp.float32),
          jax.ShapeDtypeStruct((di, B), jnp.float32),
      ],
      mesh=mesh,
      scratch_types=[
          pltpu.VMEM((du, bw), jnp.int32),
          pltpu.VMEM((di, bw), jnp.int32),
          pltpu.VMEM((du, bw), jnp.float32),
          pltpu.VMEM((di, bw), jnp.float32),
          pltpu.SemaphoreType.DMA,
          pltpu.SemaphoreType.DMA,
      ],
      compiler_params=pltpu.CompilerParams(use_tc_tiling_on_sc=False),
  )
  def k(uf, itf, ju, ji, uo, io, ju_v, ji_v, ub, ib, su, si):
    c = lax.axis_index("c")
    s = lax.axis_index("s")
    wid = s * _NC + c
    base = wid * bw
    pltpu.sync_copy(ju.at[:, pl.ds(base, bw)], ju_v)
    pltpu.sync_copy(ji.at[:, pl.ds(base, bw)], ji_v)
    cps = []
    for r in range(du):
      cps.append(pltpu.async_copy(uf.at[ju_v.at[r]], ub.at[r], su))
    for r in range(di):
      cps.append(pltpu.async_copy(itf.at[ji_v.at[r]], ib.at[r], si))
    for cp in cps:
      cp.wait()
    pltpu.sync_copy(ub, uo.at[:, pl.ds(base, bw)])
    pltpu.sync_copy(ib, io.at[:, pl.ds(base, bw)])

  return k(uflat, iflat, idxu, idxi)


def _leaky(x):
  return jnp.where(x >= 0, x, 0.01 * x)


def _tc_mlp_t(uT, iT, featsT, pgT, cgT, inT, pgTt, cgTt, inTt,
              w1_parts, WsT, bsT, *, interpret=False):
  B = uT.shape[1]
  BM = 1024
  grid = (B // BM,)
  n_pg = pgTt.shape[1]
  n_cg = cgTt.shape[1]
  n_in = inTt.shape[1]

  w1u, w1i, w1pg, w1cg, w1in, w1f = w1_parts

  def body(u_ref, i_ref, f_ref, pg_ref, cg_ref, in_ref,
           pgt_ref, cgt_ref, int_ref,
           w1u_ref, w1i_ref, w1pg_ref, w1cg_ref, w1in_ref, w1f_ref,
           *wb_refs):
    o_ref = wb_refs[-1]
    w_refs = wb_refs[0:5]
    b_refs = wb_refs[5:11]

    def mm(a, b):
      return jnp.dot(a, b, preferred_element_type=jnp.float32)

    oh_pg = (lax.broadcasted_iota(jnp.int32, (n_pg, 1), 0) == pg_ref[...]
             ).astype(jnp.float32)
    oh_cg = (lax.broadcasted_iota(jnp.int32, (n_cg, 1), 0) == cg_ref[...]
             ).astype(jnp.float32)
    oh_in = (lax.broadcasted_iota(jnp.int32, (n_in, 1), 0) == in_ref[...]
             ).astype(jnp.float32)
    pgE = mm(pgt_ref[...], oh_pg)
    cgE = mm(cgt_ref[...], oh_cg)
    inE = mm(int_ref[...], oh_in)
    x = (mm(w1u_ref[...], u_ref[...]) + mm(w1i_ref[...], i_ref[...])
         + mm(w1pg_ref[...], pgE) + mm(w1cg_ref[...], cgE)
         + mm(w1in_ref[...], inE) + mm(w1f_ref[...], f_ref[...])
         + b_refs[0][...])
    x = _leaky(x)
    for wr, br in zip(w_refs, b_refs[1:]):
      x = mm(wr[...], x) + br[...]
      x = _leaky(x)
    o_ref[...] = x

  def col_spec(d):
    return pl.BlockSpec((d, BM), lambda i: (0, i))

  def full_spec(shape):
    return pl.BlockSpec(shape, lambda i: (0, 0))

  in_specs = [
      col_spec(uT.shape[0]), col_spec(iT.shape[0]), col_spec(featsT.shape[0]),
      col_spec(1), col_spec(1), col_spec(1),
      full_spec(pgTt.shape), full_spec(cgTt.shape), full_spec(inTt.shape),
      full_spec(w1u.shape), full_spec(w1i.shape), full_spec(w1pg.shape),
      full_spec(w1cg.shape), full_spec(w1in.shape), full_spec(w1f.shape),
  ]
  for W in WsT:
    in_specs.append(full_spec(W.shape))
  for b in bsT:
    in_specs.append(full_spec(b.shape))

  out_dim = WsT[-1].shape[0]
  return pl.pallas_call(
      body,
      grid=grid,
      in_specs=in_specs,
      out_specs=pl.BlockSpec((out_dim, BM), lambda i: (0, i)),
      out_shape=jax.ShapeDtypeStruct((out_dim, B), jnp.float32),
      interpret=interpret,
  )(uT, iT, featsT, pgT, cgT, inT, pgTt, cgTt, inTt,
    w1u, w1i, w1pg, w1cg, w1in, w1f, *WsT, *bsT)


def kernel(user_input, item_input, prices, sales_channels, club_status,
           age_groups, product_groups, color_groups, index_name,
           user_table, item_table, pg_table, cg_table, in_table, Ws, bs):
  B = user_input.shape[0]
  nu, du = user_table.shape
  ni, di = item_table.shape
  ui32 = user_input.astype(jnp.int32)
  ii32 = item_input.astype(jnp.int32)

  idxu = ui32[None, :] + (jnp.arange(du, dtype=jnp.int32) * nu)[:, None]
  idxi = ii32[None, :] + (jnp.arange(di, dtype=jnp.int32) * ni)[:, None]
  uflat = user_table.T.reshape(-1)
  iflat = item_table.T.reshape(-1)
  uT, iT = _sc_gather_t(uflat, iflat, idxu, idxi, du, di)

  W1 = Ws[0]
  w1_parts = (W1[0:16].T, W1[16:48].T, W1[48:58].T, W1[58:66].T,
              W1[66:72].T, W1[72:76].T)
  WsT = [W.T for W in Ws[1:]]
  bsT = [b.reshape(-1, 1) for b in bs]
  featsT = jnp.stack([prices, sales_channels, club_status, age_groups], axis=0)
  pgT = product_groups.astype(jnp.int32).reshape(1, B)
  cgT = color_groups.astype(jnp.int32).reshape(1, B)
  inT = index_name.astype(jnp.int32).reshape(1, B)
  outT = _tc_mlp_t(uT, iT, featsT, pgT, cgT, inT,
                   pg_table.T, cg_table.T, in_table.T,
                   w1_parts, WsT, bsT)
  return outT.T


# speedup vs baseline: 5.1982x; 5.1982x over previous
import functools

import jax
import jax.numpy as jnp
from jax import lax
from jax.experimental import pallas as pl
from jax.experimental.pallas import tpu as pltpu
from jax.experimental.pallas import tpu_sc as plsc

_NC = 2
_NS = 16
_NW = _NC * _NS


def _sc_gather_t(utP, itP, uidx, iidx):
  B = uidx.shape[0]
  bw = B // _NW
  du = utP.shape[0]
  di = itP.shape[0]
  lanes = 16
  lane_ids = None

  mesh = plsc.VectorSubcoreMesh(core_axis_name="c", subcore_axis_name="s")

  KU = 8
  KI = 4

  @functools.partial(
      pl.kernel,
      out_type=[
          jax.ShapeDtypeStruct((du, B), jnp.float32),
          jax.ShapeDtypeStruct((di, B), jnp.float32),
      ],
      mesh=mesh,
      scratch_types=[
          pltpu.VMEM((bw,), jnp.int32),
          pltpu.VMEM((bw,), jnp.int32),
          pltpu.VMEM((2, KU, du, 128), jnp.float32),
          pltpu.VMEM((2, KI, di, 128), jnp.float32),
          pltpu.VMEM((du, bw), jnp.float32),
          pltpu.VMEM((di, bw), jnp.float32),
          pltpu.SemaphoreType.DMA,
          pltpu.SemaphoreType.DMA,
          pltpu.SemaphoreType.DMA,
          pltpu.SemaphoreType.DMA,
      ],
      compiler_params=pltpu.CompilerParams(
          use_tc_tiling_on_sc=True, needs_layout_passes=False),
  )
  def k(ut, it, ju, ji, uo, io, ju_v, ji_v, ubufs, ibufs,
        ubT, ibT, su0, su1, si0, si1):
    c = lax.axis_index("c")
    s = lax.axis_index("s")
    wid = s * _NC + c
    base = wid * bw
    pltpu.sync_copy(ju.at[pl.ds(base, bw)], ju_v)
    pltpu.sync_copy(ji.at[pl.ds(base, bw)], ji_v)
    lane_ids = lax.iota(jnp.int32, lanes)

    def pick(vec, lane):
      return jnp.max(jnp.where(lane_ids == lane, vec, 0))

    def run_table(tab, jv, bufs, sems, stage, d, K):
      sub = d // lanes
      nbat = 16 // K

      def issue(grp, lb, sel):
        for t in range(K):
          off = pl.multiple_of((pick(grp, lb + t) >> 7) * 128, 128)
          pltpu.async_copy(tab.at[:, pl.ds(off, 128)], bufs.at[sel, t],
                           sems[sel])

      def drain(sel):
        for t in range(K):
          pltpu.make_async_copy(tab.at[:, pl.ds(0, 128)], bufs.at[sel, t],
                                sems[sel]).wait()

      def extract(grp, g, lb, sel):
        lvec = grp & 127
        selv = jnp.full((lanes,), sel, jnp.int32)
        for t in range(K):
          l = jnp.full((lanes,), pick(lvec, lb + t), jnp.int32)
          rfull = jnp.full((lanes,), g * 16 + lb + t, jnp.int32)
          tv = jnp.full((lanes,), t, jnp.int32)
          for q in range(sub):
            rows = lane_ids + q * lanes
            col = plsc.load_gather(bufs, [selv, tv, rows, l])
            plsc.store_scatter(stage, [rows, rfull], col)

      def group2(g, carry):
        grp = jv[pl.ds(pl.multiple_of(g * 16, 16), 16)]
        issue(grp, 0, 0)
        for bidx in range(nbat):
          sel = bidx % 2
          if bidx + 1 < nbat:
            issue(grp, (bidx + 1) * K, 1 - sel)
          drain(sel)
          extract(grp, g, bidx * K, sel)
        return carry

      lax.fori_loop(0, bw // 16, group2, 0)

    run_table(ut, ju_v, ubufs, (su0, su1), ubT, du, KU)
    run_table(it, ji_v, ibufs, (si0, si1), ibT, di, KI)
    pltpu.sync_copy(ubT, uo.at[:, pl.ds(base, bw)])
    pltpu.sync_copy(ibT, io.at[:, pl.ds(base, bw)])

  return k(utP, itP, uidx, iidx)


def _leaky(x):
  return jnp.where(x >= 0, x, 0.01 * x)


def _tc_mlp_t(uT, iT, featsT, pgT, cgT, inT, pgTt, cgTt, inTt,
              w1_parts, WsT, bsT, *, interpret=False):
  B = uT.shape[1]
  BM = 1024
  grid = (B // BM,)
  n_pg = pgTt.shape[1]
  n_cg = cgTt.shape[1]
  n_in = inTt.shape[1]

  w1u, w1i, w1pg, w1cg, w1in, w1f = w1_parts

  def body(u_ref, i_ref, f_ref, pg_ref, cg_ref, in_ref,
           pgt_ref, cgt_ref, int_ref,
           w1u_ref, w1i_ref, w1pg_ref, w1cg_ref, w1in_ref, w1f_ref,
           *wb_refs):
    o_ref = wb_refs[-1]
    w_refs = wb_refs[0:5]
    b_refs = wb_refs[5:11]

    def mm(a, b):
      return jnp.dot(a, b, preferred_element_type=jnp.float32)

    oh_pg = (lax.broadcasted_iota(jnp.int32, (n_pg, 1), 0) == pg_ref[...]
             ).astype(jnp.float32)
    oh_cg = (lax.broadcasted_iota(jnp.int32, (n_cg, 1), 0) == cg_ref[...]
             ).astype(jnp.float32)
    oh_in = (lax.broadcasted_iota(jnp.int32, (n_in, 1), 0) == in_ref[...]
             ).astype(jnp.float32)
    pgE = mm(pgt_ref[...], oh_pg)
    cgE = mm(cgt_ref[...], oh_cg)
    inE = mm(int_ref[...], oh_in)
    x = (mm(w1u_ref[...], u_ref[...]) + mm(w1i_ref[...], i_ref[...])
         + mm(w1pg_ref[...], pgE) + mm(w1cg_ref[...], cgE)
         + mm(w1in_ref[...], inE) + mm(w1f_ref[...], f_ref[...])
         + b_refs[0][...])
    x = _leaky(x)
    for wr, br in zip(w_refs, b_refs[1:]):
      x = mm(wr[...], x) + br[...]
      x = _leaky(x)
    o_ref[...] = x

  def col_spec(d):
    return pl.BlockSpec((d, BM), lambda i: (0, i))

  def full_spec(shape):
    return pl.BlockSpec(shape, lambda i: (0, 0))

  in_specs = [
      col_spec(uT.shape[0]), col_spec(iT.shape[0]), col_spec(featsT.shape[0]),
      col_spec(1), col_spec(1), col_spec(1),
      full_spec(pgTt.shape), full_spec(cgTt.shape), full_spec(inTt.shape),
      full_spec(w1u.shape), full_spec(w1i.shape), full_spec(w1pg.shape),
      full_spec(w1cg.shape), full_spec(w1in.shape), full_spec(w1f.shape),
  ]
  for W in WsT:
    in_specs.append(full_spec(W.shape))
  for b in bsT:
    in_specs.append(full_spec(b.shape))

  out_dim = WsT[-1].shape[0]
  return pl.pallas_call(
      body,
      grid=grid,
      in_specs=in_specs,
      out_specs=pl.BlockSpec((out_dim, BM), lambda i: (0, i)),
      out_shape=jax.ShapeDtypeStruct((out_dim, B), jnp.float32),
      interpret=interpret,
  )(uT, iT, featsT, pgT, cgT, inT, pgTt, cgTt, inTt,
    w1u, w1i, w1pg, w1cg, w1in, w1f, *WsT, *bsT)


def kernel(user_input, item_input, prices, sales_channels, club_status,
           age_groups, product_groups, color_groups, index_name,
           user_table, item_table, pg_table, cg_table, in_table, Ws, bs):
  B = user_input.shape[0]
  nu, du = user_table.shape
  ni, di = item_table.shape
  ui32 = user_input.astype(jnp.int32)
  ii32 = item_input.astype(jnp.int32)

  uT, iT = _sc_gather_t(user_table.T, item_table.T, ui32, ii32)

  W1 = Ws[0]
  w1_parts = (W1[0:16].T, W1[16:48].T, W1[48:58].T, W1[58:66].T,
              W1[66:72].T, W1[72:76].T)
  WsT = [W.T for W in Ws[1:]]
  bsT = [b.reshape(-1, 1) for b in bs]
  featsT = jnp.stack([prices, sales_channels, club_status, age_groups], axis=0)
  pgT = product_groups.astype(jnp.int32).reshape(1, B)
  cgT = color_groups.astype(jnp.int32).reshape(1, B)
  inT = index_name.astype(jnp.int32).reshape(1, B)
  outT = _tc_mlp_t(uT, iT, featsT, pgT, cgT, inT,
                   pg_table.T, cg_table.T, in_table.T,
                   w1_parts, WsT, bsT)
  return outT.T


# speedup vs baseline: 5.3141x; 1.0223x over previous
import functools

import jax
import jax.numpy as jnp
from jax import lax
from jax.experimental import pallas as pl
from jax.experimental.pallas import tpu as pltpu
from jax.experimental.pallas import tpu_sc as plsc

_NC = 2
_NS = 16
_NW = _NC * _NS


def _sc_gather_t(utP, itP, uidx, iidx):
  B = uidx.shape[0]
  bw = B // _NW
  du = utP.shape[0]
  di = itP.shape[0]
  lanes = 16
  lane_ids = None

  mesh = plsc.VectorSubcoreMesh(core_axis_name="c", subcore_axis_name="s")

  KU = 8
  KI = 4

  @functools.partial(
      pl.kernel,
      out_type=[
          jax.ShapeDtypeStruct((du, B), jnp.float32),
          jax.ShapeDtypeStruct((di, B), jnp.float32),
      ],
      mesh=mesh,
      scratch_types=[
          pltpu.VMEM((bw,), jnp.int32),
          pltpu.VMEM((bw,), jnp.int32),
          pltpu.VMEM((2, KU, du, 128), jnp.float32),
          pltpu.VMEM((2, KI, di, 128), jnp.float32),
          pltpu.VMEM((du, bw), jnp.float32),
          pltpu.VMEM((di, bw), jnp.float32),
          pltpu.SemaphoreType.DMA,
          pltpu.SemaphoreType.DMA,
          pltpu.SemaphoreType.DMA,
          pltpu.SemaphoreType.DMA,
      ],
      compiler_params=pltpu.CompilerParams(
          use_tc_tiling_on_sc=True, needs_layout_passes=False),
  )
  def k(ut, it, ju, ji, uo, io, ju_v, ji_v, ubufs, ibufs,
        ubT, ibT, su0, su1, si0, si1):
    c = lax.axis_index("c")
    s = lax.axis_index("s")
    wid = s * _NC + c
    base = wid * bw
    pltpu.sync_copy(ju.at[pl.ds(base, bw)], ju_v)
    pltpu.sync_copy(ji.at[pl.ds(base, bw)], ji_v)
    lane_ids = lax.iota(jnp.int32, lanes)

    def pick(vec, lane):
      return jnp.max(jnp.where(lane_ids == lane, vec, 0))

    def run_table(tab, jv, bufs, sems, stage, d, K):
      sub = d // lanes
      nbat = 16 // K

      def issue(grp, lb, sel):
        for t in range(K):
          off = pl.multiple_of((pick(grp, lb + t) >> 7) * 128, 128)
          pltpu.async_copy(tab.at[:, pl.ds(off, 128)], bufs.at[sel, t],
                           sems[sel])

      def drain(sel):
        for t in range(K):
          pltpu.make_async_copy(tab.at[:, pl.ds(0, 128)], bufs.at[sel, t],
                                sems[sel]).wait()

      def extract(grp, g, lb, sel):
        lvec = grp & 127
        selv = jnp.full((lanes,), sel, jnp.int32)
        for t in range(K):
          l = jnp.full((lanes,), pick(lvec, lb + t), jnp.int32)
          rfull = jnp.full((lanes,), g * 16 + lb + t, jnp.int32)
          tv = jnp.full((lanes,), t, jnp.int32)
          for q in range(sub):
            rows = lane_ids + q * lanes
            col = plsc.load_gather(bufs, [selv, tv, rows, l])
            plsc.store_scatter(stage, [rows, rfull], col)

      return issue, drain, extract

    iss_u, drn_u, ext_u = run_table(ut, ju_v, ubufs, (su0, su1), ubT, du, KU)
    iss_i, drn_i, ext_i = run_table(it, ji_v, ibufs, (si0, si1), ibT, di, KI)
    nb_u = 16 // KU
    nb_i = 16 // KI

    def group2(g, carry):
      grpU = ju_v[pl.ds(pl.multiple_of(g * 16, 16), 16)]
      grpI = ji_v[pl.ds(pl.multiple_of(g * 16, 16), 16)]
      iss_u(grpU, 0, 0)
      iss_i(grpI, 0, 0)
      for bidx in range(max(nb_u, nb_i)):
        sel = bidx % 2
        if bidx + 1 < nb_u:
          iss_u(grpU, (bidx + 1) * KU, 1 - sel)
        if bidx + 1 < nb_i:
          iss_i(grpI, (bidx + 1) * KI, 1 - sel)
        if bidx < nb_u:
          drn_u(sel)
          ext_u(grpU, g, bidx * KU, sel)
        if bidx < nb_i:
          drn_i(sel)
          ext_i(grpI, g, bidx * KI, sel)
      return carry

    lax.fori_loop(0, bw // 16, group2, 0)
    pltpu.sync_copy(ubT, uo.at[:, pl.ds(base, bw)])
    pltpu.sync_copy(ibT, io.at[:, pl.ds(base, bw)])

  return k(utP, itP, uidx, iidx)


def _leaky(x):
  return jnp.where(x >= 0, x, 0.01 * x)


def _tc_mlp_t(uT, iT, featsT, pgT, cgT, inT, pgTt, cgTt, inTt,
              w1_parts, WsT, bsT, *, interpret=False):
  B = uT.shape[1]
  BM = 1024
  grid = (B // BM,)
  n_pg = pgTt.shape[1]
  n_cg = cgTt.shape[1]
  n_in = inTt.shape[1]

  w1u, w1i, w1pg, w1cg, w1in, w1f = w1_parts

  def body(u_ref, i_ref, f_ref, pg_ref, cg_ref, in_ref,
           pgt_ref, cgt_ref, int_ref,
           w1u_ref, w1i_ref, w1pg_ref, w1cg_ref, w1in_ref, w1f_ref,
           *wb_refs):
    o_ref = wb_refs[-1]
    w_refs = wb_refs[0:5]
    b_refs = wb_refs[5:11]

    def mm(a, b):
      return jnp.dot(a, b, preferred_element_type=jnp.float32)

    oh_pg = (lax.broadcasted_iota(jnp.int32, (n_pg, 1), 0) == pg_ref[...]
             ).astype(jnp.float32)
    oh_cg = (lax.broadcasted_iota(jnp.int32, (n_cg, 1), 0) == cg_ref[...]
             ).astype(jnp.float32)
    oh_in = (lax.broadcasted_iota(jnp.int32, (n_in, 1), 0) == in_ref[...]
             ).astype(jnp.float32)
    pgE = mm(pgt_ref[...], oh_pg)
    cgE = mm(cgt_ref[...], oh_cg)
    inE = mm(int_ref[...], oh_in)
    x = (mm(w1u_ref[...], u_ref[...]) + mm(w1i_ref[...], i_ref[...])
         + mm(w1pg_ref[...], pgE) + mm(w1cg_ref[...], cgE)
         + mm(w1in_ref[...], inE) + mm(w1f_ref[...], f_ref[...])
         + b_refs[0][...])
    x = _leaky(x)
    for wr, br in zip(w_refs, b_refs[1:]):
      x = mm(wr[...], x) + br[...]
      x = _leaky(x)
    o_ref[...] = x

  def col_spec(d):
    return pl.BlockSpec((d, BM), lambda i: (0, i))

  def full_spec(shape):
    return pl.BlockSpec(shape, lambda i: (0, 0))

  in_specs = [
      col_spec(uT.shape[0]), col_spec(iT.shape[0]), col_spec(featsT.shape[0]),
      col_spec(1), col_spec(1), col_spec(1),
      full_spec(pgTt.shape), full_spec(cgTt.shape), full_spec(inTt.shape),
      full_spec(w1u.shape), full_spec(w1i.shape), full_spec(w1pg.shape),
      full_spec(w1cg.shape), full_spec(w1in.shape), full_spec(w1f.shape),
  ]
  for W in WsT:
    in_specs.append(full_spec(W.shape))
  for b in bsT:
    in_specs.append(full_spec(b.shape))

  out_dim = WsT[-1].shape[0]
  return pl.pallas_call(
      body,
      grid=grid,
      in_specs=in_specs,
      out_specs=pl.BlockSpec((out_dim, BM), lambda i: (0, i)),
      out_shape=jax.ShapeDtypeStruct((out_dim, B), jnp.float32),
      interpret=interpret,
  )(uT, iT, featsT, pgT, cgT, inT, pgTt, cgTt, inTt,
    w1u, w1i, w1pg, w1cg, w1in, w1f, *WsT, *bsT)


def kernel(user_input, item_input, prices, sales_channels, club_status,
           age_groups, product_groups, color_groups, index_name,
           user_table, item_table, pg_table, cg_table, in_table, Ws, bs):
  B = user_input.shape[0]
  nu, du = user_table.shape
  ni, di = item_table.shape
  ui32 = user_input.astype(jnp.int32)
  ii32 = item_input.astype(jnp.int32)

  uT, iT = _sc_gather_t(user_table.T, item_table.T, ui32, ii32)

  W1 = Ws[0]
  w1_parts = (W1[0:16].T, W1[16:48].T, W1[48:58].T, W1[58:66].T,
              W1[66:72].T, W1[72:76].T)
  WsT = [W.T for W in Ws[1:]]
  bsT = [b.reshape(-1, 1) for b in bs]
  featsT = jnp.stack([prices, sales_channels, club_status, age_groups], axis=0)
  pgT = product_groups.astype(jnp.int32).reshape(1, B)
  cgT = color_groups.astype(jnp.int32).reshape(1, B)
  inT = index_name.astype(jnp.int32).reshape(1, B)
  outT = _tc_mlp_t(uT, iT, featsT, pgT, cgT, inT,
                   pg_table.T, cg_table.T, in_table.T,
                   w1_parts, WsT, bsT)
  return outT.T


# speedup vs baseline: 8.0049x; 1.5064x over previous
import functools

import jax
import jax.numpy as jnp
from jax import lax
from jax.experimental import pallas as pl
from jax.experimental.pallas import tpu as pltpu
from jax.experimental.pallas import tpu_sc as plsc

_NC = 2
_NS = 16
_NW = _NC * _NS


def _sc_gather_user(utP, uidx):
  B = uidx.shape[0]
  bw = B // _NW
  du = utP.shape[0]
  lanes = 16
  K = 8

  mesh = plsc.VectorSubcoreMesh(core_axis_name="c", subcore_axis_name="s")

  @functools.partial(
      pl.kernel,
      out_type=jax.ShapeDtypeStruct((du, B), jnp.float32),
      mesh=mesh,
      scratch_types=[
          pltpu.VMEM((bw,), jnp.int32),
          pltpu.VMEM((2, K, du, 128), jnp.float32),
          pltpu.VMEM((du, bw), jnp.float32),
          pltpu.SemaphoreType.DMA,
          pltpu.SemaphoreType.DMA,
      ],
      compiler_params=pltpu.CompilerParams(
          use_tc_tiling_on_sc=True, needs_layout_passes=False),
  )
  def k(ut, ju, uo, ju_v, bufs, ubT, s0, s1):
    c = lax.axis_index("c")
    s = lax.axis_index("s")
    wid = s * _NC + c
    base = wid * bw
    pltpu.sync_copy(ju.at[pl.ds(base, bw)], ju_v)
    lane_ids = lax.iota(jnp.int32, lanes)
    sems = (s0, s1)

    def pick(vec, lane):
      return jnp.max(jnp.where(lane_ids == lane, vec, 0))

    def issue(grp, lb, sel):
      for t in range(K):
        off = pl.multiple_of((pick(grp, lb + t) >> 7) * 128, 128)
        pltpu.async_copy(ut.at[:, pl.ds(off, 128)], bufs.at[sel, t],
                         sems[sel])

    def drain(sel):
      for t in range(K):
        pltpu.make_async_copy(ut.at[:, pl.ds(0, 128)], bufs.at[sel, t],
                              sems[sel]).wait()

    def extract(grp, g, lb, sel):
      lvec = grp & 127
      selv = jnp.full((lanes,), sel, jnp.int32)
      for t in range(K):
        l = jnp.full((lanes,), pick(lvec, lb + t), jnp.int32)
        rfull = jnp.full((lanes,), g * 16 + lb + t, jnp.int32)
        tv = jnp.full((lanes,), t, jnp.int32)
        col = plsc.load_gather(bufs, [selv, tv, lane_ids, l])
        plsc.store_scatter(ubT, [lane_ids, rfull], col)

    def group(g, carry):
      grp = ju_v[pl.ds(pl.multiple_of(g * 16, 16), 16)]
      issue(grp, 0, 0)
      issue(grp, K, 1)
      drain(0)
      extract(grp, g, 0, 0)
      drain(1)
      extract(grp, g, K, 1)
      return carry

    lax.fori_loop(0, bw // 16, group, 0)
    pltpu.sync_copy(ubT, uo.at[:, pl.ds(base, bw)])

  return k(utP, uidx)


def _sc_gather_item(it2, iidx):
  B = iidx.shape[0]
  bw = B // _NW
  di = 32
  CH = 128
  nch = bw // CH
  lanes = 16

  mesh = plsc.VectorSubcoreMesh(core_axis_name="c", subcore_axis_name="s")

  @functools.partial(
      pl.kernel,
      out_type=jax.ShapeDtypeStruct((di, B), jnp.float32),
      mesh=mesh,
      scratch_types=[
          pltpu.VMEM((bw,), jnp.int32),
          pltpu.VMEM((nch, CH), jnp.int32),
          pltpu.VMEM((2, CH, 128), jnp.float32),
          pltpu.VMEM((di, bw), jnp.float32),
          pltpu.SemaphoreType.DMA,
          pltpu.SemaphoreType.DMA,
      ],
      compiler_params=pltpu.CompilerParams(
          use_tc_tiling_on_sc=True, needs_layout_passes=False),
  )
  def k(it, ji, io, jv, jrow, bufs, ibT, s0, s1):
    c = lax.axis_index("c")
    s = lax.axis_index("s")
    wid = s * _NC + c
    base = wid * bw
    pltpu.sync_copy(ji.at[pl.ds(base, bw)], jv)
    lane_ids = lax.iota(jnp.int32, lanes)
    sems = (s0, s1)

    for j in range(nch):
      for u in range(CH // lanes):
        v = jv[pl.ds(j * CH + u * lanes, lanes)]
        jrow[j, pl.ds(u * lanes, lanes)] = v >> 2

    cps = {}
    def issue(j, sel):
      cps[sel] = pltpu.async_copy(it.at[jrow.at[j]], bufs.at[sel], sems[sel])

    def extract(j, sel):
      selv = jnp.full((lanes,), sel, jnp.int32)
      def grp8(g8, carry):
        st = j * CH + g8 * lanes
        idxv = jv[pl.ds(pl.multiple_of(st, 16), lanes)]
        sub32 = (idxv & 3) * 32
        rloc = lane_ids + g8 * lanes
        rglob = lane_ids + st
        for q in range(di):
          col = plsc.load_gather(bufs, [selv, rloc, sub32 + q])
          plsc.store_scatter(ibT, [jnp.full((lanes,), q, jnp.int32), rglob],
                             col)
        return carry
      lax.fori_loop(0, CH // lanes, grp8, 0)

    issue(0, 0)
    for j in range(nch):
      sel = j % 2
      if j + 1 < nch:
        issue(j + 1, 1 - sel)
      cps[sel].wait()
      extract(j, sel)
    pltpu.sync_copy(ibT, io.at[:, pl.ds(base, bw)])

  return k(it2, iidx)


def _leaky(x):
  return jnp.where(x >= 0, x, 0.01 * x)


def _tc_mlp_t(uT, iT, featsT, pgT, cgT, inT, pgTt, cgTt, inTt,
              w1_parts, WsT, bsT, *, interpret=False):
  B = uT.shape[1]
  BM = 1024
  grid = (B // BM,)
  n_pg = pgTt.shape[1]
  n_cg = cgTt.shape[1]
  n_in = inTt.shape[1]

  w1u, w1i, w1pg, w1cg, w1in, w1f = w1_parts

  def body(u_ref, i_ref, f_ref, pg_ref, cg_ref, in_ref,
           pgt_ref, cgt_ref, int_ref,
           w1u_ref, w1i_ref, w1pg_ref, w1cg_ref, w1in_ref, w1f_ref,
           *wb_refs):
    o_ref = wb_refs[-1]
    w_refs = wb_refs[0:5]
    b_refs = wb_refs[5:11]

    def mm(a, b):
      return jnp.dot(a, b, preferred_element_type=jnp.float32)

    oh_pg = (lax.broadcasted_iota(jnp.int32, (n_pg, 1), 0) == pg_ref[...]
             ).astype(jnp.float32)
    oh_cg = (lax.broadcasted_iota(jnp.int32, (n_cg, 1), 0) == cg_ref[...]
             ).astype(jnp.float32)
    oh_in = (lax.broadcasted_iota(jnp.int32, (n_in, 1), 0) == in_ref[...]
             ).astype(jnp.float32)
    pgE = mm(pgt_ref[...], oh_pg)
    cgE = mm(cgt_ref[...], oh_cg)
    inE = mm(int_ref[...], oh_in)
    x = (mm(w1u_ref[...], u_ref[...]) + mm(w1i_ref[...], i_ref[...])
         + mm(w1pg_ref[...], pgE) + mm(w1cg_ref[...], cgE)
         + mm(w1in_ref[...], inE) + mm(w1f_ref[...], f_ref[...])
         + b_refs[0][...])
    x = _leaky(x)
    for wr, br in zip(w_refs, b_refs[1:]):
      x = mm(wr[...], x) + br[...]
      x = _leaky(x)
    o_ref[...] = x

  def col_spec(d):
    return pl.BlockSpec((d, BM), lambda i: (0, i))

  def full_spec(shape):
    return pl.BlockSpec(shape, lambda i: (0, 0))

  in_specs = [
      col_spec(uT.shape[0]), col_spec(iT.shape[0]), col_spec(featsT.shape[0]),
      col_spec(1), col_spec(1), col_spec(1),
      full_spec(pgTt.shape), full_spec(cgTt.shape), full_spec(inTt.shape),
      full_spec(w1u.shape), full_spec(w1i.shape), full_spec(w1pg.shape),
      full_spec(w1cg.shape), full_spec(w1in.shape), full_spec(w1f.shape),
  ]
  for W in WsT:
    in_specs.append(full_spec(W.shape))
  for b in bsT:
    in_specs.append(full_spec(b.shape))

  out_dim = WsT[-1].shape[0]
  return pl.pallas_call(
      body,
      grid=grid,
      in_specs=in_specs,
      out_specs=pl.BlockSpec((out_dim, BM), lambda i: (0, i)),
      out_shape=jax.ShapeDtypeStruct((out_dim, B), jnp.float32),
      interpret=interpret,
  )(uT, iT, featsT, pgT, cgT, inT, pgTt, cgTt, inTt,
    w1u, w1i, w1pg, w1cg, w1in, w1f, *WsT, *bsT)


def kernel(user_input, item_input, prices, sales_channels, club_status,
           age_groups, product_groups, color_groups, index_name,
           user_table, item_table, pg_table, cg_table, in_table, Ws, bs):
  B = user_input.shape[0]
  nu, du = user_table.shape
  ni, di = item_table.shape
  ui32 = user_input.astype(jnp.int32)
  ii32 = item_input.astype(jnp.int32)

  uT = _sc_gather_user(user_table.T, ui32)
  iT = _sc_gather_item(item_table.reshape(-1, 128), ii32)

  W1 = Ws[0]
  w1_parts = (W1[0:16].T, W1[16:48].T, W1[48:58].T, W1[58:66].T,
              W1[66:72].T, W1[72:76].T)
  WsT = [W.T for W in Ws[1:]]
  bsT = [b.reshape(-1, 1) for b in bs]
  featsT = jnp.stack([prices, sales_channels, club_status, age_groups], axis=0)
  pgT = product_groups.astype(jnp.int32).reshape(1, B)
  cgT = color_groups.astype(jnp.int32).reshape(1, B)
  inT = index_name.astype(jnp.int32).reshape(1, B)
  outT = _tc_mlp_t(uT, iT, featsT, pgT, cgT, inT,
                   pg_table.T, cg_table.T, in_table.T,
                   w1_parts, WsT, bsT)
  return outT.T


# speedup vs baseline: 8.5470x; 1.0677x over previous
import functools

import jax
import jax.numpy as jnp
from jax import lax
from jax.experimental import pallas as pl
from jax.experimental.pallas import tpu as pltpu
from jax.experimental.pallas import tpu_sc as plsc

_NC = 2
_NS = 16
_NW = _NC * _NS


def _sc_gather_user(utP, uidx):
  B = uidx.shape[0]
  bw = B // _NW
  du = utP.shape[0]
  lanes = 16
  K = 16

  mesh = plsc.VectorSubcoreMesh(core_axis_name="c", subcore_axis_name="s")

  @functools.partial(
      pl.kernel,
      out_type=jax.ShapeDtypeStruct((du, B), jnp.float32),
      mesh=mesh,
      scratch_types=[
          pltpu.VMEM((bw,), jnp.int32),
          pltpu.VMEM((2, K, du, 128), jnp.float32),
          pltpu.VMEM((du, bw), jnp.float32),
          pltpu.SemaphoreType.DMA,
          pltpu.SemaphoreType.DMA,
      ],
      compiler_params=pltpu.CompilerParams(
          use_tc_tiling_on_sc=True, needs_layout_passes=False),
  )
  def k(ut, ju, uo, ju_v, bufs, ubT, s0, s1):
    c = lax.axis_index("c")
    s = lax.axis_index("s")
    wid = s * _NC + c
    base = wid * bw
    pltpu.sync_copy(ju.at[pl.ds(base, bw)], ju_v)
    lane_ids = lax.iota(jnp.int32, lanes)
    sems = (s0, s1)

    def pick(vec, lane):
      return jnp.max(jnp.where(lane_ids == lane, vec, 0))

    ng = bw // 16

    def load_grp(g):
      return ju_v[pl.ds(pl.multiple_of(g * 16, 16), 16)]

    def issue(grp, sel):
      for t in range(K):
        off = pl.multiple_of((pick(grp, t) >> 7) * 128, 128)
        pltpu.async_copy(ut.at[:, pl.ds(off, 128)], bufs.at[sel, t],
                         sems[sel])

    def drain(sel):
      for t in range(K):
        pltpu.make_async_copy(ut.at[:, pl.ds(0, 128)], bufs.at[sel, t],
                              sems[sel]).wait()

    def extract(grp, g, sel):
      lvec = grp & 127
      selv = jnp.full((lanes,), sel, jnp.int32)
      for t in range(K):
        l = jnp.full((lanes,), pick(lvec, t), jnp.int32)
        rfull = jnp.full((lanes,), g * 16 + t, jnp.int32)
        tv = jnp.full((lanes,), t, jnp.int32)
        col = plsc.load_gather(bufs, [selv, tv, lane_ids, l])
        plsc.store_scatter(ubT, [lane_ids, rfull], col)

    issue(load_grp(0), 0)
    issue(load_grp(1), 1)

    def pair(p, carry):
      g = 2 * p
      drain(0)
      extract(load_grp(g), g, 0)

      @pl.when(g + 2 < ng)
      def _():
        issue(load_grp(g + 2), 0)

      drain(1)
      extract(load_grp(g + 1), g + 1, 1)

      @pl.when(g + 3 < ng)
      def _():
        issue(load_grp(g + 3), 1)

      return carry

    lax.fori_loop(0, ng // 2, pair, 0)
    pltpu.sync_copy(ubT, uo.at[:, pl.ds(base, bw)])

  return k(utP, uidx)


def _sc_gather_item(it2, iidx):
  B = iidx.shape[0]
  bw = B // _NW
  di = 32
  CH = 128
  nch = bw // CH
  lanes = 16

  mesh = plsc.VectorSubcoreMesh(core_axis_name="c", subcore_axis_name="s")

  @functools.partial(
      pl.kernel,
      out_type=jax.ShapeDtypeStruct((di, B), jnp.float32),
      mesh=mesh,
      scratch_types=[
          pltpu.VMEM((bw,), jnp.int32),
          pltpu.VMEM((nch, CH), jnp.int32),
          pltpu.VMEM((2, CH, 128), jnp.float32),
          pltpu.VMEM((di, bw), jnp.float32),
          pltpu.SemaphoreType.DMA,
          pltpu.SemaphoreType.DMA,
      ],
      compiler_params=pltpu.CompilerParams(
          use_tc_tiling_on_sc=True, needs_layout_passes=False),
  )
  def k(it, ji, io, jv, jrow, bufs, ibT, s0, s1):
    c = lax.axis_index("c")
    s = lax.axis_index("s")
    wid = s * _NC + c
    base = wid * bw
    pltpu.sync_copy(ji.at[pl.ds(base, bw)], jv)
    lane_ids = lax.iota(jnp.int32, lanes)
    sems = (s0, s1)

    for j in range(nch):
      for u in range(CH // lanes):
        v = jv[pl.ds(j * CH + u * lanes, lanes)]
        jrow[j, pl.ds(u * lanes, lanes)] = v >> 2

    cps = {}
    def issue(j, sel):
      cps[sel] = pltpu.async_copy(it.at[jrow.at[j]], bufs.at[sel], sems[sel])

    def extract(j, sel):
      selv = jnp.full((lanes,), sel, jnp.int32)
      def grp8(g8, carry):
        st = j * CH + g8 * lanes
        idxv = jv[pl.ds(pl.multiple_of(st, 16), lanes)]
        sub32 = (idxv & 3) * 32
        rloc = lane_ids + g8 * lanes
        rglob = lane_ids + st
        for q in range(di):
          col = plsc.load_gather(bufs, [selv, rloc, sub32 + q])
          plsc.store_scatter(ibT, [jnp.full((lanes,), q, jnp.int32), rglob],
                             col)
        return carry
      lax.fori_loop(0, CH // lanes, grp8, 0)

    issue(0, 0)
    for j in range(nch):
      sel = j % 2
      if j + 1 < nch:
        issue(j + 1, 1 - sel)
      cps[sel].wait()
      extract(j, sel)
    pltpu.sync_copy(ibT, io.at[:, pl.ds(base, bw)])

  return k(it2, iidx)


def _leaky(x):
  return jnp.where(x >= 0, x, 0.01 * x)


def _tc_mlp_t(uT, iT, featsT, pgT, cgT, inT, pgTt, cgTt, inTt,
              w1_parts, WsT, bsT, *, interpret=False):
  B = uT.shape[1]
  BM = 1024
  grid = (B // BM,)
  n_pg = pgTt.shape[1]
  n_cg = cgTt.shape[1]
  n_in = inTt.shape[1]

  w1u, w1i, w1pg, w1cg, w1in, w1f = w1_parts

  def body(u_ref, i_ref, f_ref, pg_ref, cg_ref, in_ref,
           pgt_ref, cgt_ref, int_ref,
           w1u_ref, w1i_ref, w1pg_ref, w1cg_ref, w1in_ref, w1f_ref,
           *wb_refs):
    o_ref = wb_refs[-1]
    w_refs = wb_refs[0:5]
    b_refs = wb_refs[5:11]

    def mm(a, b):
      return jnp.dot(a, b, preferred_element_type=jnp.float32)

    oh_pg = (lax.broadcasted_iota(jnp.int32, (n_pg, 1), 0) == pg_ref[...]
             ).astype(jnp.float32)
    oh_cg = (lax.broadcasted_iota(jnp.int32, (n_cg, 1), 0) == cg_ref[...]
             ).astype(jnp.float32)
    oh_in = (lax.broadcasted_iota(jnp.int32, (n_in, 1), 0) == in_ref[...]
             ).astype(jnp.float32)
    pgE = mm(pgt_ref[...], oh_pg)
    cgE = mm(cgt_ref[...], oh_cg)
    inE = mm(int_ref[...], oh_in)
    x = (mm(w1u_ref[...], u_ref[...]) + mm(w1i_ref[...], i_ref[...])
         + mm(w1pg_ref[...], pgE) + mm(w1cg_ref[...], cgE)
         + mm(w1in_ref[...], inE) + mm(w1f_ref[...], f_ref[...])
         + b_refs[0][...])
    x = _leaky(x)
    for wr, br in zip(w_refs, b_refs[1:]):
      x = mm(wr[...], x) + br[...]
      x = _leaky(x)
    o_ref[...] = x

  def col_spec(d):
    return pl.BlockSpec((d, BM), lambda i: (0, i))

  def full_spec(shape):
    return pl.BlockSpec(shape, lambda i: (0, 0))

  in_specs = [
      col_spec(uT.shape[0]), col_spec(iT.shape[0]), col_spec(featsT.shape[0]),
      col_spec(1), col_spec(1), col_spec(1),
      full_spec(pgTt.shape), full_spec(cgTt.shape), full_spec(inTt.shape),
      full_spec(w1u.shape), full_spec(w1i.shape), full_spec(w1pg.shape),
      full_spec(w1cg.shape), full_spec(w1in.shape), full_spec(w1f.shape),
  ]
  for W in WsT:
    in_specs.append(full_spec(W.shape))
  for b in bsT:
    in_specs.append(full_spec(b.shape))

  out_dim = WsT[-1].shape[0]
  return pl.pallas_call(
      body,
      grid=grid,
      in_specs=in_specs,
      out_specs=pl.BlockSpec((out_dim, BM), lambda i: (0, i)),
      out_shape=jax.ShapeDtypeStruct((out_dim, B), jnp.float32),
      interpret=interpret,
  )(uT, iT, featsT, pgT, cgT, inT, pgTt, cgTt, inTt,
    w1u, w1i, w1pg, w1cg, w1in, w1f, *WsT, *bsT)


def kernel(user_input, item_input, prices, sales_channels, club_status,
           age_groups, product_groups, color_groups, index_name,
           user_table, item_table, pg_table, cg_table, in_table, Ws, bs):
  B = user_input.shape[0]
  nu, du = user_table.shape
  ni, di = item_table.shape
  ui32 = user_input.astype(jnp.int32)
  ii32 = item_input.astype(jnp.int32)

  uT = _sc_gather_user(user_table.T, ui32)
  iT = _sc_gather_item(item_table.reshape(-1, 128), ii32)

  W1 = Ws[0]
  w1_parts = (W1[0:16].T, W1[16:48].T, W1[48:58].T, W1[58:66].T,
              W1[66:72].T, W1[72:76].T)
  WsT = [W.T for W in Ws[1:]]
  bsT = [b.reshape(-1, 1) for b in bs]
  featsT = jnp.stack([prices, sales_channels, club_status, age_groups], axis=0)
  pgT = product_groups.astype(jnp.int32).reshape(1, B)
  cgT = color_groups.astype(jnp.int32).reshape(1, B)
  inT = index_name.astype(jnp.int32).reshape(1, B)
  outT = _tc_mlp_t(uT, iT, featsT, pgT, cgT, inT,
                   pg_table.T, cg_table.T, in_table.T,
                   w1_parts, WsT, bsT)
  return outT.T
